# baseline (speedup 1.0000x reference)
import functools

import jax
import jax.numpy as jnp
from jax import lax
from jax.experimental import pallas as pl
from jax.experimental.pallas import tpu as pltpu
from jax.experimental.pallas import tpu_sc as plsc

WD = 0.00025
B = 16384
D = 128
NC = 2
NS = 16
L = 16
NW = NC * NS
BPW = B // NW
CH = 128
NCH = BPW // CH

_mesh = plsc.VectorSubcoreMesh(
    core_axis_name="c", subcore_axis_name="s", num_cores=NC, num_subcores=NS
)


@functools.partial(
    pl.kernel,
    out_type=(
        jax.ShapeDtypeStruct((B, L), jnp.float32),
        jax.ShapeDtypeStruct((NW, L), jnp.float32),
    ),
    mesh=_mesh,
    scratch_types=[
        pltpu.VMEM((NCH, CH), jnp.int32),
        pltpu.VMEM((NCH, CH), jnp.int32),
        pltpu.VMEM((NCH, CH), jnp.int32),
        pltpu.VMEM((CH, D), jnp.float32),
        pltpu.VMEM((CH, D), jnp.float32),
        pltpu.VMEM((CH, D), jnp.float32),
        pltpu.VMEM((CH, D), jnp.float32),
        pltpu.VMEM((CH, D), jnp.float32),
        pltpu.VMEM((CH, D), jnp.float32),
        pltpu.VMEM((CH, L), jnp.float32),
        pltpu.VMEM((L,), jnp.float32),
        pltpu.SemaphoreType.DMA,
        pltpu.SemaphoreType.DMA,
        pltpu.SemaphoreType.DMA,
        pltpu.SemaphoreType.DMA,
        pltpu.SemaphoreType.DMA,
        pltpu.SemaphoreType.DMA,
    ],
)
def _sc_bpr(u_hbm, i_hbm, j_hbm, w_hbm, h_hbm, xp_hbm, reg_hbm,
            uix, iix, jix, ue0, ie0, je0, ue1, ie1, je1, xa, regv,
            s0, s1, s2, s3, s4, s5):
    wid = lax.axis_index("s") * NC + lax.axis_index("c")
    base = wid * BPW

    idx_cps = []
    for c in range(NCH):
        idx_cps.append(pltpu.async_copy(u_hbm.at[pl.ds(base + c * CH, CH)], uix.at[c], s0))
        idx_cps.append(pltpu.async_copy(i_hbm.at[pl.ds(base + c * CH, CH)], iix.at[c], s1))
        idx_cps.append(pltpu.async_copy(j_hbm.at[pl.ds(base + c * CH, CH)], jix.at[c], s2))
    for cp in idx_cps:
        cp.wait()

    banks = ((ue0, ie0, je0, s0, s1, s2), (ue1, ie1, je1, s3, s4, s5))

    def fire(c):
        ub, ib, jb, sa, sb, sc = banks[c & 1]
        return (pltpu.async_copy(w_hbm.at[uix.at[c]], ub, sa),
                pltpu.async_copy(h_hbm.at[iix.at[c]], ib, sb),
                pltpu.async_copy(h_hbm.at[jix.at[c]], jb, sc))

    pend = fire(0)
    ru = jnp.zeros((L,), jnp.float32)
    rij = jnp.zeros((L,), jnp.float32)
    for c in range(NCH):
        nxt = fire(c + 1) if c + 1 < NCH else None
        for cp in pend:
            cp.wait()
        ub, ib, jb = banks[c & 1][:3]

        def row(r, carry):
            cru, crij = carry
            acc = jnp.zeros((L,), jnp.float32)
            for k in range(D // L):
                a = ub[r, pl.ds(k * L, L)]
                b = ib[r, pl.ds(k * L, L)]
                g = jb[r, pl.ds(k * L, L)]
                acc = acc + a * (b - g)
                cru = cru + a * a
                crij = crij + b * b + g * g
            xa[r] = acc
            return (cru, crij)

        ru, rij = lax.fori_loop(0, CH, row, (ru, rij))
        pltpu.sync_copy(xa, xp_hbm.at[pl.ds(base + c * CH, CH)])
        pend = nxt

    regv[...] = ru + rij
    pltpu.sync_copy(regv, reg_hbm.at[wid])


def _tc_copyw_body(x_ref, o_ref, sem):
    pltpu.async_copy(x_ref, o_ref, sem).wait()


_tc_copy_w = pl.pallas_call(
    _tc_copyw_body,
    in_specs=[pl.BlockSpec(memory_space=pl.ANY)],
    out_specs=pl.BlockSpec(memory_space=pl.ANY),
    out_shape=jax.ShapeDtypeStruct((100000, D), jnp.float32),
    scratch_shapes=[pltpu.SemaphoreType.DMA],
)


def _tc_copyh_body(h_ref, xp_ref, reg_ref, wdep_ref, ho_ref, o_ref, sem):
    del wdep_ref
    cp = pltpu.async_copy(h_ref, ho_ref, sem)
    x = jnp.sum(xp_ref[...], axis=1)
    ls = jnp.minimum(x, 0.0) - jnp.log1p(jnp.exp(-jnp.abs(x)))
    total = -jnp.sum(ls) + WD * jnp.sum(reg_ref[...])
    o_ref[...] = jnp.reshape(total, (1, 1))
    cp.wait()


_tc_copyh_final = pl.pallas_call(
    _tc_copyh_body,
    grid=(1,),
    in_specs=[
        pl.BlockSpec(memory_space=pl.ANY),
        pl.BlockSpec((B, L), lambda g: (0, 0)),
        pl.BlockSpec((NW, L), lambda g: (0, 0)),
        pl.BlockSpec((8, D), lambda g: (0, 0)),
    ],
    out_specs=[
        pl.BlockSpec(memory_space=pl.ANY),
        pl.BlockSpec((1, 1), lambda g: (0, 0)),
    ],
    out_shape=[
        jax.ShapeDtypeStruct((100000, D), jnp.float32),
        jax.ShapeDtypeStruct((1, 1), jnp.float32),
    ],
    scratch_shapes=[pltpu.SemaphoreType.DMA],
)


def kernel(u, i, j, W, H):
    u32 = u.astype(jnp.int32)
    i32 = i.astype(jnp.int32)
    j32 = j.astype(jnp.int32)
    xp, regp = _sc_bpr(u32, i32, j32, W, H)
    w_out = _tc_copy_w(W)
    h_out, loss2d = _tc_copyh_final(H, xp, regp, w_out)
    return (loss2d.reshape(()), w_out, h_out)


# speedup vs baseline: 29.3677x; 29.3677x over previous
import functools

import jax
import jax.numpy as jnp
from jax import lax
from jax.experimental import pallas as pl
from jax.experimental.pallas import tpu as pltpu
from jax.experimental.pallas import tpu_sc as plsc

WD = 0.00025
B = 16384
D = 128
NC = 2
NS = 16
L = 16
NW = NC * NS
BPW = B // NW
CH = 128
NCH = BPW // CH

_mesh = plsc.VectorSubcoreMesh(
    core_axis_name="c", subcore_axis_name="s", num_cores=NC, num_subcores=NS
)


@functools.partial(
    pl.kernel,
    out_type=(
        jax.ShapeDtypeStruct((B, L), jnp.float32),
        jax.ShapeDtypeStruct((NW, L), jnp.float32),
    ),
    mesh=_mesh,
    scratch_types=[
        pltpu.VMEM((NCH, CH), jnp.int32),
        pltpu.VMEM((NCH, CH), jnp.int32),
        pltpu.VMEM((NCH, CH), jnp.int32),
        pltpu.VMEM((CH, D), jnp.float32),
        pltpu.VMEM((CH, D), jnp.float32),
        pltpu.VMEM((CH, D), jnp.float32),
        pltpu.VMEM((CH, D), jnp.float32),
        pltpu.VMEM((CH, D), jnp.float32),
        pltpu.VMEM((CH, D), jnp.float32),
        pltpu.VMEM((CH, L), jnp.float32),
        pltpu.VMEM((L,), jnp.float32),
        pltpu.SemaphoreType.DMA,
        pltpu.SemaphoreType.DMA,
        pltpu.SemaphoreType.DMA,
        pltpu.SemaphoreType.DMA,
        pltpu.SemaphoreType.DMA,
        pltpu.SemaphoreType.DMA,
    ],
)
def _sc_bpr(u_hbm, i_hbm, j_hbm, w_hbm, h_hbm, xp_hbm, reg_hbm,
            uix, iix, jix, ue0, ie0, je0, ue1, ie1, je1, xa, regv,
            s0, s1, s2, s3, s4, s5):
    wid = lax.axis_index("s") * NC + lax.axis_index("c")
    base = wid * BPW

    idx_cps = []
    for c in range(NCH):
        idx_cps.append(pltpu.async_copy(u_hbm.at[pl.ds(base + c * CH, CH)], uix.at[c], s0))
        idx_cps.append(pltpu.async_copy(i_hbm.at[pl.ds(base + c * CH, CH)], iix.at[c], s1))
        idx_cps.append(pltpu.async_copy(j_hbm.at[pl.ds(base + c * CH, CH)], jix.at[c], s2))
    for cp in idx_cps:
        cp.wait()

    banks = ((ue0, ie0, je0, s0, s1, s2), (ue1, ie1, je1, s3, s4, s5))

    def fire(c):
        ub, ib, jb, sa, sb, sc = banks[c & 1]
        return (pltpu.async_copy(w_hbm.at[uix.at[c]], ub, sa),
                pltpu.async_copy(h_hbm.at[iix.at[c]], ib, sb),
                pltpu.async_copy(h_hbm.at[jix.at[c]], jb, sc))

    pend = fire(0)
    ru = jnp.zeros((L,), jnp.float32)
    rij = jnp.zeros((L,), jnp.float32)
    for c in range(NCH):
        nxt = fire(c + 1) if c + 1 < NCH else None
        for cp in pend:
            cp.wait()
        ub, ib, jb = banks[c & 1][:3]

        def row(r, carry):
            cru, crij = carry
            acc = jnp.zeros((L,), jnp.float32)
            for k in range(D // L):
                a = ub[r, pl.ds(k * L, L)]
                b = ib[r, pl.ds(k * L, L)]
                g = jb[r, pl.ds(k * L, L)]
                acc = acc + a * (b - g)
                cru = cru + a * a
                crij = crij + b * b + g * g
            xa[r] = acc
            return (cru, crij)

        ru, rij = lax.fori_loop(0, CH, row, (ru, rij))
        pltpu.sync_copy(xa, xp_hbm.at[pl.ds(base + c * CH, CH)])
        pend = nxt

    regv[...] = ru + rij
    pltpu.sync_copy(regv, reg_hbm.at[wid])


_HG = 4
_HBLK = 100000 // _HG
_XBLK = B // _HG


def _tc_copyh_body(h_ref, xp_ref, reg_ref, wdep_ref, ho_ref, o_ref):
    del wdep_ref
    ho_ref[...] = h_ref[...]
    g = pl.program_id(0)

    @pl.when(g == 0)
    def _():
        o_ref[...] = jnp.reshape(WD * jnp.sum(reg_ref[...]), (1, 1))

    x = jnp.sum(xp_ref[...], axis=1)
    ls = jnp.minimum(x, 0.0) - jnp.log1p(jnp.exp(-jnp.abs(x)))
    o_ref[...] += jnp.reshape(-jnp.sum(ls), (1, 1))


_tc_copyh_final = pl.pallas_call(
    _tc_copyh_body,
    grid=(_HG,),
    in_specs=[
        pl.BlockSpec((_HBLK, D), lambda g: (g, 0)),
        pl.BlockSpec((_XBLK, L), lambda g: (g, 0)),
        pl.BlockSpec((NW, L), lambda g: (0, 0)),
        pl.BlockSpec((8, D), lambda g: (0, 0)),
    ],
    out_specs=[
        pl.BlockSpec((_HBLK, D), lambda g: (g, 0)),
        pl.BlockSpec((1, 1), lambda g: (0, 0)),
    ],
    out_shape=[
        jax.ShapeDtypeStruct((100000, D), jnp.float32),
        jax.ShapeDtypeStruct((1, 1), jnp.float32),
    ],
)


def kernel(u, i, j, W, H):
    u32 = u.astype(jnp.int32)
    i32 = i.astype(jnp.int32)
    j32 = j.astype(jnp.int32)
    xp, regp = _sc_bpr(u32, i32, j32, W, H)
    w_out = jnp.copy(W)
    h_out, loss2d = _tc_copyh_final(H, xp, regp, w_out)
    return (loss2d.reshape(()), w_out, h_out)


# speedup vs baseline: 30.5637x; 1.0407x over previous
import functools

import jax
import jax.numpy as jnp
from jax import lax
from jax.experimental import pallas as pl
from jax.experimental.pallas import tpu as pltpu
from jax.experimental.pallas import tpu_sc as plsc

WD = 0.00025
B = 16384
D = 128
NC = 2
NS = 16
L = 16
NW = NC * NS
BPW = B // NW
CH = 128
NCH = BPW // CH

_mesh = plsc.VectorSubcoreMesh(
    core_axis_name="c", subcore_axis_name="s", num_cores=NC, num_subcores=NS
)


@functools.partial(
    pl.kernel,
    out_type=(
        jax.ShapeDtypeStruct((B, L), jnp.float32),
        jax.ShapeDtypeStruct((NW, L), jnp.float32),
    ),
    mesh=_mesh,
    scratch_types=[
        pltpu.VMEM((NCH, CH), jnp.int32),
        pltpu.VMEM((NCH, CH), jnp.int32),
        pltpu.VMEM((NCH, CH), jnp.int32),
        pltpu.VMEM((CH, D), jnp.float32),
        pltpu.VMEM((CH, D), jnp.float32),
        pltpu.VMEM((CH, D), jnp.float32),
        pltpu.VMEM((CH, D), jnp.float32),
        pltpu.VMEM((CH, D), jnp.float32),
        pltpu.VMEM((CH, D), jnp.float32),
        pltpu.VMEM((CH, L), jnp.float32),
        pltpu.VMEM((L,), jnp.float32),
        pltpu.SemaphoreType.DMA,
        pltpu.SemaphoreType.DMA,
        pltpu.SemaphoreType.DMA,
        pltpu.SemaphoreType.DMA,
        pltpu.SemaphoreType.DMA,
        pltpu.SemaphoreType.DMA,
    ],
)
def _sc_bpr(u_hbm, i_hbm, j_hbm, w_hbm, h_hbm, xp_hbm, reg_hbm,
            uix, iix, jix, ue0, ie0, je0, ue1, ie1, je1, xa, regv,
            s0, s1, s2, s3, s4, s5):
    wid = lax.axis_index("s") * NC + lax.axis_index("c")
    base = wid * BPW

    idx_cps = []
    for c in range(NCH):
        idx_cps.append(pltpu.async_copy(u_hbm.at[pl.ds(base + c * CH, CH)], uix.at[c], s0))
        idx_cps.append(pltpu.async_copy(i_hbm.at[pl.ds(base + c * CH, CH)], iix.at[c], s1))
        idx_cps.append(pltpu.async_copy(j_hbm.at[pl.ds(base + c * CH, CH)], jix.at[c], s2))
    for cp in idx_cps:
        cp.wait()

    banks = ((ue0, ie0, je0, s0, s1, s2), (ue1, ie1, je1, s3, s4, s5))

    def fire(c):
        ub, ib, jb, sa, sb, sc = banks[c & 1]
        return (pltpu.async_copy(w_hbm.at[uix.at[c]], ub, sa),
                pltpu.async_copy(h_hbm.at[iix.at[c]], ib, sb),
                pltpu.async_copy(h_hbm.at[jix.at[c]], jb, sc))

    pend = fire(0)
    ru = jnp.zeros((L,), jnp.float32)
    rij = jnp.zeros((L,), jnp.float32)
    for c in range(NCH):
        nxt = fire(c + 1) if c + 1 < NCH else None
        for cp in pend:
            cp.wait()
        ub, ib, jb = banks[c & 1][:3]

        def row(r, carry):
            cru, crij = carry
            acc = jnp.zeros((L,), jnp.float32)
            for k in range(D // L):
                a = ub[r, pl.ds(k * L, L)]
                b = ib[r, pl.ds(k * L, L)]
                g = jb[r, pl.ds(k * L, L)]
                acc = acc + a * (b - g)
                cru = cru + a * a
                crij = crij + b * b + g * g
            xa[r] = acc
            return (cru, crij)

        ru, rij = lax.fori_loop(0, CH, row, (ru, rij))
        pltpu.sync_copy(xa, xp_hbm.at[pl.ds(base + c * CH, CH)])
        pend = nxt

    regv[...] = ru + rij
    pltpu.sync_copy(regv, reg_hbm.at[wid])


_CPR = 5000


def _tc_copy_body(x_ref, o_ref):
    o_ref[...] = x_ref[...]


_tc_copy = pl.pallas_call(
    _tc_copy_body,
    grid=(100000 // _CPR,),
    in_specs=[pl.BlockSpec((_CPR, D), lambda g: (g, 0))],
    out_specs=pl.BlockSpec((_CPR, D), lambda g: (g, 0)),
    out_shape=jax.ShapeDtypeStruct((100000, D), jnp.float32),
)


_HG = 20
_HBLK = 100000 // _HG
_XG = 8
_XBLK = B // _XG


def _tc_copyh_body(h_ref, xp_ref, reg_ref, wdep_ref, ho_ref, o_ref):
    del wdep_ref
    ho_ref[...] = h_ref[...]
    g = pl.program_id(0)

    @pl.when(g == 0)
    def _():
        o_ref[...] = jnp.reshape(WD * jnp.sum(reg_ref[...]), (1, 1))

    @pl.when(g < _XG)
    def _():
        x = jnp.sum(xp_ref[...], axis=1)
        ls = jnp.minimum(x, 0.0) - jnp.log1p(jnp.exp(-jnp.abs(x)))
        o_ref[...] += jnp.reshape(-jnp.sum(ls), (1, 1))


_tc_copyh_final = pl.pallas_call(
    _tc_copyh_body,
    grid=(_HG,),
    in_specs=[
        pl.BlockSpec((_HBLK, D), lambda g: (g, 0)),
        pl.BlockSpec((_XBLK, L), lambda g: (jnp.minimum(g, _XG - 1), 0)),
        pl.BlockSpec((NW, L), lambda g: (0, 0)),
        pl.BlockSpec((8, D), lambda g: (0, 0)),
    ],
    out_specs=[
        pl.BlockSpec((_HBLK, D), lambda g: (g, 0)),
        pl.BlockSpec((1, 1), lambda g: (0, 0)),
    ],
    out_shape=[
        jax.ShapeDtypeStruct((100000, D), jnp.float32),
        jax.ShapeDtypeStruct((1, 1), jnp.float32),
    ],
)


def kernel(u, i, j, W, H):
    u32 = u.astype(jnp.int32)
    i32 = i.astype(jnp.int32)
    j32 = j.astype(jnp.int32)
    xp, regp = _sc_bpr(u32, i32, j32, W, H)
    w_out = _tc_copy(W)
    h_out, loss2d = _tc_copyh_final(H, xp, regp, w_out)
    return (loss2d.reshape(()), w_out, h_out)


# speedup vs baseline: 32.5652x; 1.0655x over previous
import functools

import jax
import jax.numpy as jnp
from jax import lax
from jax.experimental import pallas as pl
from jax.experimental.pallas import tpu as pltpu
from jax.experimental.pallas import tpu_sc as plsc

WD = 0.00025
B = 16384
D = 128
NC = 2
NS = 16
L = 16
NW = NC * NS
BPW = B // NW
CH = 128
NCH = BPW // CH

_mesh = plsc.VectorSubcoreMesh(
    core_axis_name="c", subcore_axis_name="s", num_cores=NC, num_subcores=NS
)


@functools.partial(
    pl.kernel,
    out_type=(
        jax.ShapeDtypeStruct((B, L), jnp.float32),
        jax.ShapeDtypeStruct((NW, L), jnp.float32),
    ),
    mesh=_mesh,
    scratch_types=[
        pltpu.VMEM((NCH, CH), jnp.int32),
        pltpu.VMEM((NCH, CH), jnp.int32),
        pltpu.VMEM((NCH, CH), jnp.int32),
        pltpu.VMEM((CH, D), jnp.float32),
        pltpu.VMEM((CH, D), jnp.float32),
        pltpu.VMEM((CH, D), jnp.float32),
        pltpu.VMEM((CH, D), jnp.float32),
        pltpu.VMEM((CH, D), jnp.float32),
        pltpu.VMEM((CH, D), jnp.float32),
        pltpu.VMEM((CH, L), jnp.float32),
        pltpu.VMEM((L,), jnp.float32),
        pltpu.SemaphoreType.DMA,
        pltpu.SemaphoreType.DMA,
        pltpu.SemaphoreType.DMA,
        pltpu.SemaphoreType.DMA,
        pltpu.SemaphoreType.DMA,
        pltpu.SemaphoreType.DMA,
    ],
)
def _sc_bpr(u_hbm, i_hbm, j_hbm, w_hbm, h_hbm, xp_hbm, reg_hbm,
            uix, iix, jix, ue0, ie0, je0, ue1, ie1, je1, xa, regv,
            s0, s1, s2, s3, s4, s5):
    wid = lax.axis_index("s") * NC + lax.axis_index("c")
    base = wid * BPW

    idx_cps = []
    for c in range(NCH):
        idx_cps.append(pltpu.async_copy(u_hbm.at[pl.ds(base + c * CH, CH)], uix.at[c], s0))
        idx_cps.append(pltpu.async_copy(i_hbm.at[pl.ds(base + c * CH, CH)], iix.at[c], s1))
        idx_cps.append(pltpu.async_copy(j_hbm.at[pl.ds(base + c * CH, CH)], jix.at[c], s2))
    for cp in idx_cps:
        cp.wait()

    banks = ((ue0, ie0, je0, s0, s1, s2), (ue1, ie1, je1, s3, s4, s5))

    def fire(c):
        ub, ib, jb, sa, sb, sc = banks[c & 1]
        return (pltpu.async_copy(w_hbm.at[uix.at[c]], ub, sa),
                pltpu.async_copy(h_hbm.at[iix.at[c]], ib, sb),
                pltpu.async_copy(h_hbm.at[jix.at[c]], jb, sc))

    pend = fire(0)
    ru = jnp.zeros((L,), jnp.float32)
    rij = jnp.zeros((L,), jnp.float32)
    for c in range(NCH):
        nxt = fire(c + 1) if c + 1 < NCH else None
        for cp in pend:
            cp.wait()
        ub, ib, jb = banks[c & 1][:3]

        def row(r, carry):
            cru, crij = carry
            acc = jnp.zeros((L,), jnp.float32)
            for k in range(D // L):
                a = ub[r, pl.ds(k * L, L)]
                b = ib[r, pl.ds(k * L, L)]
                g = jb[r, pl.ds(k * L, L)]
                acc = acc + a * (b - g)
                cru = cru + a * a
                crij = crij + b * b + g * g
            xa[r] = acc
            return (cru, crij)

        ru, rij = lax.fori_loop(0, CH, row, (ru, rij))
        pltpu.sync_copy(xa, xp_hbm.at[pl.ds(base + c * CH, CH)])
        pend = nxt

    regv[...] = ru + rij
    pltpu.sync_copy(regv, reg_hbm.at[wid])


_RB = 4
_RR = 5000
_RN = 100000 // _RR


def _tc_copy_body(x_hbm, o_hbm, b0, b1, b2, b3,
                  si0, si1, si2, si3, so0, so1, so2, so3):
    bufs = (b0, b1, b2, b3)
    sin = (si0, si1, si2, si3)
    sout = (so0, so1, so2, so3)
    cin = {}
    cout = {}
    for k in range(_RB):
        cin[k] = pltpu.async_copy(x_hbm.at[pl.ds(k * _RR, _RR)], bufs[k], sin[k])
    for k in range(_RN):
        b = k % _RB
        cin[k].wait()
        cout[k] = pltpu.async_copy(bufs[b], o_hbm.at[pl.ds(k * _RR, _RR)], sout[b])
        nk = k + _RB
        if nk < _RN:
            cout[k].wait()
            cin[nk] = pltpu.async_copy(x_hbm.at[pl.ds(nk * _RR, _RR)], bufs[b], sin[b])
    for k in range(_RN - _RB, _RN):
        cout[k].wait()


_tc_copy = pl.pallas_call(
    _tc_copy_body,
    in_specs=[pl.BlockSpec(memory_space=pl.ANY)],
    out_specs=pl.BlockSpec(memory_space=pl.ANY),
    out_shape=jax.ShapeDtypeStruct((100000, D), jnp.float32),
    scratch_shapes=(
        [pltpu.VMEM((_RR, D), jnp.float32)] * _RB
        + [pltpu.SemaphoreType.DMA] * (2 * _RB)
    ),
)


_HG = 4
_HBLK = 100000 // _HG
_XBLK = B // _HG


def _tc_copyh_body(h_ref, xp_ref, reg_ref, wdep_ref, ho_ref, o_ref):
    del wdep_ref
    ho_ref[...] = h_ref[...]
    g = pl.program_id(0)

    @pl.when(g == 0)
    def _():
        o_ref[...] = jnp.reshape(WD * jnp.sum(reg_ref[...]), (1, 1))

    x = jnp.sum(xp_ref[...], axis=1)
    ls = jnp.minimum(x, 0.0) - jnp.log1p(jnp.exp(-jnp.abs(x)))
    o_ref[...] += jnp.reshape(-jnp.sum(ls), (1, 1))


_tc_copyh_final = pl.pallas_call(
    _tc_copyh_body,
    grid=(_HG,),
    in_specs=[
        pl.BlockSpec((_HBLK, D), lambda g: (g, 0)),
        pl.BlockSpec((_XBLK, L), lambda g: (g, 0)),
        pl.BlockSpec((NW, L), lambda g: (0, 0)),
        pl.BlockSpec((8, D), lambda g: (0, 0)),
    ],
    out_specs=[
        pl.BlockSpec((_HBLK, D), lambda g: (g, 0)),
        pl.BlockSpec((1, 1), lambda g: (0, 0)),
    ],
    out_shape=[
        jax.ShapeDtypeStruct((100000, D), jnp.float32),
        jax.ShapeDtypeStruct((1, 1), jnp.float32),
    ],
)


def kernel(u, i, j, W, H):
    u32 = u.astype(jnp.int32)
    i32 = i.astype(jnp.int32)
    j32 = j.astype(jnp.int32)
    xp, regp = _sc_bpr(u32, i32, j32, W, H)
    w_out = _tc_copy(W)
    h_out, loss2d = _tc_copyh_final(H, xp, regp, w_out)
    return (loss2d.reshape(()), w_out, h_out)
